# baseline (speedup 1.0000x reference)
import functools

import jax
import jax.numpy as jnp
from jax import lax
from jax.experimental import pallas as pl
from jax.experimental.pallas import tpu as pltpu
from jax.experimental.pallas import tpu_sc as plsc

_B, _C, _H, _W = 2, 96, 224, 224
_K = 9
_HW = _H * _W
_L = 16
_NW = 32
_PX = 16
_NCORNER = 4 * _K
_ROWS = _NCORNER * _PX
_GCOPY = 6
_GROWS = _ROWS // _GCOPY
_CHUNKS_PER_IMG = _HW // _PX
_CHUNKS = _B * _CHUNKS_PER_IMG
_CPT = _CHUNKS // _NW
_CV = _C // _L


def _floor(v):
    vt = v.astype(jnp.int32)
    vtf = vt.astype(jnp.float32)
    gt = vtf > v
    return jnp.where(gt, vt - 1, vt), jnp.where(gt, vtf - 1.0, vtf)


_TG = _HW // 16
_TGN = _TG // _PX


def _sc_body(inp, off, msk, out, tbl, offs_v, msk_v, idx_v, w_v, g_v, pacc_v,
             acc_v, tin_v, tout_v, gsem0, gsem1, psem0, psem1, osem0, osem1):
    cix = lax.axis_index("c")
    six = lax.axis_index("s")
    wid = cix * 16 + six
    base = wid * _CPT
    lanes_i = lax.iota(jnp.int32, _L)
    lanes_f = lanes_i.astype(jnp.float32)
    lanes48 = lanes_i * 48
    sems = (gsem0, gsem1)
    psems = (psem0, psem1)
    osems = (osem0, osem1)

    tpix0 = six * _TG

    def t_fire(g, slot):
        pltpu.async_copy(inp.at[cix, :, pl.ds(tpix0 + g * _PX, _PX)],
                         tin_v.at[slot], sems[slot])

    def t_work(g, slot):
        pltpu.make_async_copy(inp.at[cix, :, pl.ds(tpix0 + g * _PX, _PX)],
                              tin_v.at[slot], sems[slot]).wait()

        @pl.when(g >= 2)
        def _():
            pltpu.make_async_copy(
                tout_v.at[slot],
                tbl.at[pl.ds(cix * _HW + tpix0 + g * _PX, _PX), :],
                osems[slot]).wait()

        for p in range(_PX):
            pvec = jnp.full((_L,), p, jnp.int32)
            for v2 in range(_CV // 2):
                ta = plsc.load_gather(tin_v.at[slot],
                                      [lanes_i + v2 * 32, pvec])
                tb = plsc.load_gather(tin_v.at[slot],
                                      [lanes_i + v2 * 32 + _L, pvec])
                packed = plsc.pack(ta, tb,
                                   format=plsc.PackFormat.INTERLEAVED,
                                   preferred_element_type=jnp.bfloat16)
                tout_v[slot, p, pl.ds(v2 * 32, 32)] = packed
        pltpu.async_copy(tout_v.at[slot],
                         tbl.at[pl.ds(cix * _HW + tpix0 + g * _PX, _PX), :],
                         osems[slot])

    t_fire(0, 0)
    t_fire(1, 1)

    def tbody(g):
        t_work(g, 0)

        @pl.when(g + 2 < _TGN)
        def _():
            t_fire(g + 2, 0)

        t_work(g + 1, 1)

        @pl.when(g + 3 < _TGN)
        def _():
            t_fire(g + 3, 1)

    pl.loop(0, _TGN, step=2)(tbody)

    for slot in range(2):
        g = _TGN - 2 + slot
        pltpu.make_async_copy(
            tout_v.at[slot],
            tbl.at[pl.ds(cix * _HW + tpix0 + g * _PX, _PX), :],
            osems[slot]).wait()

    plsc.subcore_barrier()


    def _loc(cid):
        b = cid // _CHUNKS_PER_IMG
        pix = (cid - b * _CHUNKS_PER_IMG) * _PX
        return b, pix

    def stage_a0(cid, slot):
        b, pix = _loc(cid)
        pltpu.async_copy(off.at[b, :, pl.ds(pix, _PX)], offs_v.at[slot],
                         psems[slot])
        pltpu.async_copy(msk.at[b, :, pl.ds(pix, _PX)], msk_v.at[slot],
                         psems[slot])

    def stage_a1(cid, slot):
        b, pix = _loc(cid)
        h = pix // _W
        w0 = pix - h * _W
        pltpu.make_async_copy(off.at[b, :, pl.ds(pix, _PX)],
                              offs_v.at[slot], psems[slot]).wait()
        pltpu.make_async_copy(msk.at[b, :, pl.ds(pix, _PX)],
                              msk_v.at[slot], psems[slot]).wait()
        hf = h.astype(jnp.float32)
        xlane = w0.astype(jnp.float32) + lanes_f
        rowb = b * _HW
        for k in range(_K):
            ky = k // 3 - 1
            kx = k % 3 - 1
            dy = offs_v[slot, 2 * k, :]
            dx = offs_v[slot, 2 * k + 1, :]
            m = msk_v[slot, k, :]
            y = dy + (hf + float(ky))
            x = dx + xlane + float(kx)
            y0i, y0f = _floor(y)
            x0i, x0f = _floor(x)
            ly = y - y0f
            lx = x - x0f
            hy = 1.0 - ly
            hx = 1.0 - lx
            y1i = y0i + 1
            x1i = x0i + 1
            vy0 = jnp.where((y0i >= 0) & (y0i <= _H - 1), 1.0, 0.0)
            vy1 = jnp.where((y1i >= 0) & (y1i <= _H - 1), 1.0, 0.0)
            vx0 = jnp.where((x0i >= 0) & (x0i <= _W - 1), 1.0, 0.0)
            vx1 = jnp.where((x1i >= 0) & (x1i <= _W - 1), 1.0, 0.0)
            wy0 = m * hy * vy0
            wy1 = m * ly * vy1
            wx0 = hx * vx0
            wx1 = lx * vx1
            y0c = jnp.clip(y0i, 0, _H - 1)
            y1c = jnp.clip(y1i, 0, _H - 1)
            x0c = jnp.clip(x0i, 0, _W - 1)
            x1c = jnp.clip(x1i, 0, _W - 1)
            r0 = rowb + y0c * _W
            r1 = rowb + y1c * _W
            corners = ((r0 + x0c, wy0 * wx0), (r0 + x1c, wy0 * wx1),
                       (r1 + x0c, wy1 * wx0), (r1 + x1c, wy1 * wx1))
            for c, (ivec, wvec) in enumerate(corners):
                j = 4 * k + c
                idx_v[slot, pl.ds(j * _L, _L)] = ivec
                plsc.store_scatter(w_v.at[slot], [lanes48 + j], wvec)
        for t in range(_GCOPY):
            pltpu.async_copy(
                tbl.at[idx_v.at[slot, pl.ds(t * _GROWS, _GROWS)]],
                g_v.at[slot, pl.ds(t * _GROWS, _GROWS), :],
                sems[slot])

    def stage_b(cid, slot):
        b, pix = _loc(cid)
        for t in range(_GCOPY):
            pltpu.make_async_copy(
                tbl.at[idx_v.at[slot, pl.ds(t * _GROWS, _GROWS)]],
                g_v.at[slot, pl.ds(t * _GROWS, _GROWS), :],
                sems[slot]).wait()

        @pl.when(cid >= base + 2)
        def _():
            pb, ppix = _loc(cid - 2)
            pltpu.make_async_copy(acc_v.at[slot],
                                  out.at[pb, :, pl.ds(ppix, _PX)],
                                  osems[slot]).wait()

        lane_consts = [jnp.full((_L,), i, jnp.int32) for i in range(_L)]

        def pbody(p):
            acc = [jnp.zeros((_L,), jnp.float32) for _ in range(_CV)]
            wvecs = [w_v[slot, pl.ds(p * 48 + t * _L, _L)] for t in range(3)]
            for j in range(_NCORNER):
                wv = wvecs[j // _L][lane_consts[j % _L]]
                row = j * _L + p
                for v2 in range(_CV // 2):
                    packed = g_v[slot, row, pl.ds(v2 * 32, 32)]
                    ga, gb = plsc.unpack(packed,
                                         format=plsc.PackFormat.INTERLEAVED,
                                         preferred_element_type=jnp.float32)
                    acc[2 * v2] = acc[2 * v2] + wv * ga
                    acc[2 * v2 + 1] = acc[2 * v2 + 1] + wv * gb
            for v in range(_CV):
                pacc_v[slot, p, pl.ds(v * _L, _L)] = acc[v]

        pl.loop(0, _PX)(pbody)
        for c in range(_C):
            cvec = jnp.full((_L,), c, jnp.int32)
            acc_v[slot, c, :] = plsc.load_gather(pacc_v.at[slot],
                                                 [lanes_i, cvec])
        pltpu.async_copy(acc_v.at[slot], out.at[b, :, pl.ds(pix, _PX)],
                         osems[slot])

    stage_a0(base, 0)
    stage_a0(base + 1, 1)
    stage_a1(base, 0)
    stage_a1(base + 1, 1)

    def gbody(g):
        @pl.when(g + 2 < _CPT)
        def _():
            stage_a0(base + g + 2, 0)

        stage_b(base + g, 0)

        @pl.when(g + 2 < _CPT)
        def _():
            stage_a1(base + g + 2, 0)

        @pl.when(g + 3 < _CPT)
        def _():
            stage_a0(base + g + 3, 1)

        stage_b(base + g + 1, 1)

        @pl.when(g + 3 < _CPT)
        def _():
            stage_a1(base + g + 3, 1)

    pl.loop(0, _CPT, step=2)(gbody)

    for slot in range(2):
        eb, epix = _loc(base + _CPT - 2 + slot)
        pltpu.make_async_copy(
            acc_v.at[slot],
            out.at[eb, :, pl.ds(epix, _PX)],
            osems[slot]).wait()


@functools.partial(
    pl.kernel,
    out_type=(jax.ShapeDtypeStruct((_B, _C, _HW), jnp.float32),
              jax.ShapeDtypeStruct((_B * _HW, _C), jnp.bfloat16)),
    mesh=plsc.VectorSubcoreMesh(core_axis_name="c", subcore_axis_name="s"),
    scratch_types=[
        pltpu.VMEM((2, 2 * _K, _PX), jnp.float32),
        pltpu.VMEM((2, _K, _PX), jnp.float32),
        pltpu.VMEM((2, _ROWS), jnp.int32),
        pltpu.VMEM((2, 48 * _PX), jnp.float32),
        pltpu.VMEM((2, _ROWS, _C), jnp.bfloat16),
        pltpu.VMEM((2, _PX, _C), jnp.float32),
        pltpu.VMEM((2, _C, _PX), jnp.float32),
        pltpu.VMEM((2, _C, _PX), jnp.float32),
        pltpu.VMEM((2, _PX, _C), jnp.bfloat16),
        pltpu.SemaphoreType.DMA,
        pltpu.SemaphoreType.DMA,
        pltpu.SemaphoreType.DMA,
        pltpu.SemaphoreType.DMA,
        pltpu.SemaphoreType.DMA,
        pltpu.SemaphoreType.DMA,
    ],
    compiler_params=pltpu.CompilerParams(use_tc_tiling_on_sc=False,
                                         needs_layout_passes=False),
)
def _dsm_sc(inp, off, msk, out, tbl, offs_v, msk_v, idx_v, w_v, g_v, pacc_v,
            acc_v, tin_v, tout_v, gsem0, gsem1, psem0, psem1, osem0, osem1):
    _sc_body(inp, off, msk, out, tbl, offs_v, msk_v, idx_v, w_v, g_v, pacc_v,
             acc_v, tin_v, tout_v, gsem0, gsem1, psem0, psem1, osem0, osem1)


@jax.jit
def kernel(input, offset, mask):
    inp2 = input.reshape(_B, _C, _HW)
    off2 = offset.reshape(_B, 2 * _K, _HW)
    msk2 = mask.reshape(_B, _K, _HW)
    rows, _ = _dsm_sc(inp2, off2, msk2)
    return rows.reshape(_B, _C, _H, _W)


# speedup vs baseline: 1.2565x; 1.2565x over previous
import functools

import jax
import jax.numpy as jnp
from jax import lax
from jax.experimental import pallas as pl
from jax.experimental.pallas import tpu as pltpu
from jax.experimental.pallas import tpu_sc as plsc

_B, _C, _H, _W = 2, 96, 224, 224
_K = 9
_HW = _H * _W
_L = 16
_NW = 32
_PX = 16
_NPAIR = 2 * _K
_ROWS = _NPAIR * _PX
_GCOPY = 3
_GROWS = _ROWS // _GCOPY
_C2 = 2 * _C
_CHUNKS_PER_IMG = _HW // _PX
_CHUNKS = _B * _CHUNKS_PER_IMG
_CPT = _CHUNKS // _NW
_CV = _C // _L


def _floor(v):
    vt = v.astype(jnp.int32)
    vtf = vt.astype(jnp.float32)
    gt = vtf > v
    return jnp.where(gt, vt - 1, vt), jnp.where(gt, vtf - 1.0, vtf)


_TG = _HW // 16
_TGN = _TG // _PX


def _sc_body(inp, off, msk, out, tbl, offs_v, msk_v, idx_v, w_v, g_v, acc_v,
             tin_v, tout_v, gsem0, gsem1, psem0, psem1, osem0, osem1,
             wsem0, wsem1):
    cix = lax.axis_index("c")
    six = lax.axis_index("s")
    wid = cix * 16 + six
    base = wid * _CPT
    lanes_i = lax.iota(jnp.int32, _L)
    lanes_f = lanes_i.astype(jnp.float32)
    lanes48 = lanes_i * 48
    sems = (gsem0, gsem1)
    psems = (psem0, psem1)
    osems = (osem0, osem1)
    wsems = (wsem0, wsem1)

    tpix0 = six * _TG

    def t_fire(g, slot):
        pltpu.async_copy(inp.at[cix, :, pl.ds(tpix0 + g * _PX, _PX)],
                         tin_v.at[slot], sems[slot])

    def _trow(g):
        return cix * _HW + tpix0 + g * _PX

    def t_out(g, slot, fire):
        r = _trow(g)
        c0 = pltpu.make_async_copy(tout_v.at[slot],
                                   tbl.at[pl.ds(r, _PX), pl.ds(0, _C)],
                                   osems[slot])
        c0.start() if fire else c0.wait()

        @pl.when(r > 0)
        def _():
            c1 = pltpu.make_async_copy(
                tout_v.at[slot],
                tbl.at[pl.ds(r - 1, _PX), pl.ds(_C, _C)],
                wsems[slot])
            c1.start() if fire else c1.wait()

        @pl.when(r == 0)
        def _():
            c2 = pltpu.make_async_copy(
                tout_v.at[slot, pl.ds(1, _PX - 1), :],
                tbl.at[pl.ds(0, _PX - 1), pl.ds(_C, _C)],
                wsems[slot])
            c2.start() if fire else c2.wait()

    def t_work(g, slot):
        pltpu.make_async_copy(inp.at[cix, :, pl.ds(tpix0 + g * _PX, _PX)],
                              tin_v.at[slot], sems[slot]).wait()

        @pl.when(g >= 2)
        def _():
            t_out(g - 2, slot, fire=False)

        for p in range(_PX):
            pvec = jnp.full((_L,), p, jnp.int32)
            for v2 in range(_CV // 2):
                ta = plsc.load_gather(tin_v.at[slot],
                                      [lanes_i + v2 * 32, pvec])
                tb = plsc.load_gather(tin_v.at[slot],
                                      [lanes_i + v2 * 32 + _L, pvec])
                packed = plsc.pack(ta, tb,
                                   format=plsc.PackFormat.INTERLEAVED,
                                   preferred_element_type=jnp.bfloat16)
                tout_v[slot, p, pl.ds(v2 * 32, 32)] = packed
        t_out(g, slot, fire=True)

    t_fire(0, 0)
    t_fire(1, 1)

    def tbody(g):
        t_work(g, 0)

        @pl.when(g + 2 < _TGN)
        def _():
            t_fire(g + 2, 0)

        t_work(g + 1, 1)

        @pl.when(g + 3 < _TGN)
        def _():
            t_fire(g + 3, 1)

    pl.loop(0, _TGN, step=2)(tbody)

    for slot in range(2):
        t_out(_TGN - 2 + slot, slot, fire=False)

    plsc.subcore_barrier()


    def _loc(cid):
        b = cid // _CHUNKS_PER_IMG
        pix = (cid - b * _CHUNKS_PER_IMG) * _PX
        return b, pix

    def stage_a0(cid, slot):
        b, pix = _loc(cid)
        pltpu.async_copy(off.at[b, :, pl.ds(pix, _PX)], offs_v.at[slot],
                         psems[slot])
        pltpu.async_copy(msk.at[b, :, pl.ds(pix, _PX)], msk_v.at[slot],
                         psems[slot])

    def stage_a1(cid, slot):
        b, pix = _loc(cid)
        h = pix // _W
        w0 = pix - h * _W
        pltpu.make_async_copy(off.at[b, :, pl.ds(pix, _PX)],
                              offs_v.at[slot], psems[slot]).wait()
        pltpu.make_async_copy(msk.at[b, :, pl.ds(pix, _PX)],
                              msk_v.at[slot], psems[slot]).wait()
        hf = h.astype(jnp.float32)
        xlane = w0.astype(jnp.float32) + lanes_f
        rowb = b * _HW
        for k in range(_K):
            ky = k // 3 - 1
            kx = k % 3 - 1
            dy = offs_v[slot, 2 * k, :]
            dx = offs_v[slot, 2 * k + 1, :]
            m = msk_v[slot, k, :]
            y = dy + (hf + float(ky))
            x = dx + xlane + float(kx)
            y0i, y0f = _floor(y)
            x0i, x0f = _floor(x)
            ly = y - y0f
            lx = x - x0f
            hy = 1.0 - ly
            hx = 1.0 - lx
            y1i = y0i + 1
            x1i = x0i + 1
            vy0 = jnp.where((y0i >= 0) & (y0i <= _H - 1), 1.0, 0.0)
            vy1 = jnp.where((y1i >= 0) & (y1i <= _H - 1), 1.0, 0.0)
            vx0 = jnp.where((x0i >= 0) & (x0i <= _W - 1), 1.0, 0.0)
            vx1 = jnp.where((x1i >= 0) & (x1i <= _W - 1), 1.0, 0.0)
            wy0 = m * hy * vy0
            wy1 = m * ly * vy1
            wx0 = hx * vx0
            wx1 = lx * vx1
            zero = jnp.zeros((_L,), jnp.float32)
            sel0 = x0i >= _W - 1
            sel1 = x1i <= 0
            wa = jnp.where(sel0, zero, wx0) + jnp.where(sel1, wx1, zero)
            wb = jnp.where(sel0, wx0, zero) + jnp.where(sel1, zero, wx1)
            y0c = jnp.clip(y0i, 0, _H - 1)
            y1c = jnp.clip(y1i, 0, _H - 1)
            bx = jnp.clip(x0i, 0, _W - 2)
            r0 = rowb + y0c * _W + bx
            r1 = rowb + y1c * _W + bx
            pairs = ((r0, wy0 * wa, wy0 * wb), (r1, wy1 * wa, wy1 * wb))
            for c, (ivec, wva, wvb) in enumerate(pairs):
                j = 2 * k + c
                idx_v[slot, pl.ds(j * _L, _L)] = ivec
                plsc.store_scatter(w_v.at[slot], [lanes48 + 2 * j], wva)
                plsc.store_scatter(w_v.at[slot], [lanes48 + 2 * j + 1], wvb)
        for t in range(_GCOPY):
            pltpu.async_copy(
                tbl.at[idx_v.at[slot, pl.ds(t * _GROWS, _GROWS)]],
                g_v.at[slot, pl.ds(t * _GROWS, _GROWS), :],
                sems[slot])

    def stage_b(cid, slot):
        for t in range(_GCOPY):
            pltpu.make_async_copy(
                tbl.at[idx_v.at[slot, pl.ds(t * _GROWS, _GROWS)]],
                g_v.at[slot, pl.ds(t * _GROWS, _GROWS), :],
                sems[slot]).wait()

        @pl.when(cid >= base + 2)
        def _():
            pltpu.make_async_copy(acc_v.at[slot],
                                  out.at[pl.ds(cid * _PX, _PX), :],
                                  osems[slot]).wait()

        lane_consts = [jnp.full((_L,), i, jnp.int32) for i in range(_L)]

        def pbody(p):
            acc = [jnp.zeros((_L,), jnp.float32) for _ in range(_CV)]
            wvecs = [w_v[slot, pl.ds(p * 48 + t * _L, _L)] for t in range(3)]
            for j in range(_NPAIR):
                wva = wvecs[(2 * j) // _L][lane_consts[(2 * j) % _L]]
                wvb = wvecs[(2 * j + 1) // _L][lane_consts[(2 * j + 1) % _L]]
                row = j * _L + p
                for v2 in range(_CV // 2):
                    pk0 = g_v[slot, row, pl.ds(v2 * 32, 32)]
                    ga, gb = plsc.unpack(pk0,
                                         format=plsc.PackFormat.INTERLEAVED,
                                         preferred_element_type=jnp.float32)
                    acc[2 * v2] = acc[2 * v2] + wva * ga
                    acc[2 * v2 + 1] = acc[2 * v2 + 1] + wva * gb
                    pk1 = g_v[slot, row, pl.ds(_C + v2 * 32, 32)]
                    gc, gd = plsc.unpack(pk1,
                                         format=plsc.PackFormat.INTERLEAVED,
                                         preferred_element_type=jnp.float32)
                    acc[2 * v2] = acc[2 * v2] + wvb * gc
                    acc[2 * v2 + 1] = acc[2 * v2 + 1] + wvb * gd
            for v in range(_CV):
                acc_v[slot, p, pl.ds(v * _L, _L)] = acc[v]

        pl.loop(0, _PX)(pbody)
        pltpu.async_copy(acc_v.at[slot], out.at[pl.ds(cid * _PX, _PX), :],
                         osems[slot])

    stage_a0(base, 0)
    stage_a0(base + 1, 1)
    stage_a1(base, 0)
    stage_a1(base + 1, 1)

    def gbody(g):
        @pl.when(g + 2 < _CPT)
        def _():
            stage_a0(base + g + 2, 0)

        stage_b(base + g, 0)

        @pl.when(g + 2 < _CPT)
        def _():
            stage_a1(base + g + 2, 0)

        @pl.when(g + 3 < _CPT)
        def _():
            stage_a0(base + g + 3, 1)

        stage_b(base + g + 1, 1)

        @pl.when(g + 3 < _CPT)
        def _():
            stage_a1(base + g + 3, 1)

    pl.loop(0, _CPT, step=2)(gbody)

    for slot in range(2):
        pltpu.make_async_copy(
            acc_v.at[slot],
            out.at[pl.ds((base + _CPT - 2 + slot) * _PX, _PX), :],
            osems[slot]).wait()


@functools.partial(
    pl.kernel,
    out_type=(jax.ShapeDtypeStruct((_B * _HW, _C), jnp.float32),
              jax.ShapeDtypeStruct((_B * _HW, _C2), jnp.bfloat16)),
    mesh=plsc.VectorSubcoreMesh(core_axis_name="c", subcore_axis_name="s"),
    scratch_types=[
        pltpu.VMEM((2, 2 * _K, _PX), jnp.float32),
        pltpu.VMEM((2, _K, _PX), jnp.float32),
        pltpu.VMEM((2, _ROWS), jnp.int32),
        pltpu.VMEM((2, 48 * _PX), jnp.float32),
        pltpu.VMEM((2, _ROWS, _C2), jnp.bfloat16),
        pltpu.VMEM((2, _PX, _C), jnp.float32),
        pltpu.VMEM((2, _C, _PX), jnp.float32),
        pltpu.VMEM((2, _PX, _C), jnp.bfloat16),
        pltpu.SemaphoreType.DMA,
        pltpu.SemaphoreType.DMA,
        pltpu.SemaphoreType.DMA,
        pltpu.SemaphoreType.DMA,
        pltpu.SemaphoreType.DMA,
        pltpu.SemaphoreType.DMA,
        pltpu.SemaphoreType.DMA,
        pltpu.SemaphoreType.DMA,
    ],
    compiler_params=pltpu.CompilerParams(use_tc_tiling_on_sc=False,
                                         needs_layout_passes=False),
)
def _dsm_sc(inp, off, msk, out, tbl, offs_v, msk_v, idx_v, w_v, g_v, acc_v,
            tin_v, tout_v, gsem0, gsem1, psem0, psem1, osem0, osem1,
            wsem0, wsem1):
    _sc_body(inp, off, msk, out, tbl, offs_v, msk_v, idx_v, w_v, g_v, acc_v,
             tin_v, tout_v, gsem0, gsem1, psem0, psem1, osem0, osem1,
             wsem0, wsem1)


@jax.jit
def kernel(input, offset, mask):
    inp2 = input.reshape(_B, _C, _HW)
    off2 = offset.reshape(_B, 2 * _K, _HW)
    msk2 = mask.reshape(_B, _K, _HW)
    rows, _ = _dsm_sc(inp2, off2, msk2)
    return rows.reshape(_B, _H, _W, _C).transpose(0, 3, 1, 2)
